# baseline (speedup 1.0000x reference)
import functools

import jax
import jax.numpy as jnp
from jax.experimental import pallas as pl
from jax.experimental.pallas import tpu as pltpu

N = 4096
D = 256
BM = 256
N_BM = N // BM


def _body(l_ref, x_ref, w02_ref, w1_ref, w2x2_ref, out_ref,
          lb_ref, xb_ref, y_ref, part_ref):
    p = pl.program_id(0)
    i = pl.program_id(1)
    rows = pl.ds(i * BM, BM)

    @pl.when((p == 0) & (i == 0))
    def _stash_x():
        xb_ref[...] = x_ref[...].astype(jnp.bfloat16)

    @pl.when(p == 0)
    def _phase0():
        l_blk = l_ref[...].astype(jnp.bfloat16)
        lb_ref[rows, :] = l_blk
        y = jnp.dot(l_blk, xb_ref[...], preferred_element_type=jnp.float32)
        y_ref[rows, :] = y.astype(jnp.bfloat16)
        part_ref[rows, :] = (
            jnp.dot(x_ref[rows, :], w02_ref[...],
                    preferred_element_type=jnp.float32)
            + jnp.dot(y, w1_ref[...], preferred_element_type=jnp.float32)
        )

    @pl.when(p == 1)
    def _phase1():
        z = jnp.dot(lb_ref[rows, :], y_ref[...],
                    preferred_element_type=jnp.float32)
        out_ref[...] = part_ref[rows, :] + jnp.dot(
            z, w2x2_ref[...], preferred_element_type=jnp.float32)


@functools.partial(jax.jit, static_argnames=())
def _graph_conv(x, k, L, weight):
    scale = jnp.asarray(k, jnp.float32) / jnp.float32(weight.shape[0])
    w0 = weight[0] * scale
    w1 = weight[1] * scale
    w2 = weight[2] * scale
    w02 = w0 - w2
    w2x2 = 2.0 * w2

    grid = (2, N_BM)
    l_spec = pl.BlockSpec((BM, N), lambda p, i: (i * (1 - p), 0))
    full_spec = pl.BlockSpec((N, D), lambda p, i: (0, 0))
    rowvec_spec = pl.BlockSpec((BM, D), lambda p, i: (i, 0))
    w_spec = pl.BlockSpec((D, D), lambda p, i: (0, 0))

    out = pl.pallas_call(
        _body,
        grid=grid,
        in_specs=[l_spec, full_spec, w_spec, w_spec, w_spec],
        out_specs=rowvec_spec,
        out_shape=jax.ShapeDtypeStruct((N, D), jnp.float32),
        scratch_shapes=[
            pltpu.VMEM((N, N), jnp.bfloat16),
            pltpu.VMEM((N, D), jnp.bfloat16),
            pltpu.VMEM((N, D), jnp.bfloat16),
            pltpu.VMEM((N, D), jnp.float32),
        ],
        compiler_params=pltpu.CompilerParams(
            dimension_semantics=("arbitrary", "arbitrary")),
    )(L, x, w02, w1, w2x2)
    return out


def kernel(x, k, L, weight):
    return _graph_conv(x, k, L, weight)


# speedup vs baseline: 1.1178x; 1.1178x over previous
import functools

import jax
import jax.numpy as jnp
from jax.experimental import pallas as pl
from jax.experimental.pallas import tpu as pltpu

N = 4096
D = 256
BM = 512
N_BM = N // BM


def _body(l_ref, xb_ref, w02_ref, w1_ref, w2x2_ref, out_ref,
          lb_ref, yb_ref):
    p = pl.program_id(0)
    i = pl.program_id(1)
    rows = pl.ds(i * BM, BM)

    @pl.when(p == 0)
    def _phase0():
        l_blk = l_ref[...].astype(jnp.bfloat16)
        lb_ref[rows, :] = l_blk
        y = jnp.dot(l_blk, xb_ref[...], preferred_element_type=jnp.float32)
        yb_ref[rows, :] = y.astype(jnp.bfloat16)

    @pl.when(p == 1)
    def _phase1():
        z = jnp.dot(lb_ref[rows, :], yb_ref[...],
                    preferred_element_type=jnp.float32)
        out_ref[...] = (
            jnp.dot(xb_ref[rows, :], w02_ref[...],
                    preferred_element_type=jnp.float32)
            + jnp.dot(yb_ref[rows, :], w1_ref[...],
                      preferred_element_type=jnp.float32)
            + jnp.dot(z, w2x2_ref[...], preferred_element_type=jnp.float32)
        )


@functools.partial(jax.jit, static_argnames=())
def _graph_conv(x, k, L, weight):
    scale = jnp.asarray(k, jnp.float32) / jnp.float32(weight.shape[0])
    w0 = weight[0] * scale
    w1 = weight[1] * scale
    w2 = weight[2] * scale
    w02 = w0 - w2
    w2x2 = 2.0 * w2
    xb = x.astype(jnp.bfloat16)

    grid = (2, N_BM)
    l_spec = pl.BlockSpec(
        (BM, N), lambda p, i: (jnp.where(p == 0, i, N_BM - 1), 0))
    full_spec = pl.BlockSpec((N, D), lambda p, i: (0, 0))
    out_spec = pl.BlockSpec(
        (BM, D), lambda p, i: (jnp.where(p == 0, 0, i), 0))
    w_spec = pl.BlockSpec((D, D), lambda p, i: (0, 0))

    out = pl.pallas_call(
        _body,
        grid=grid,
        in_specs=[l_spec, full_spec, w_spec, w_spec, w_spec],
        out_specs=out_spec,
        out_shape=jax.ShapeDtypeStruct((N, D), jnp.float32),
        scratch_shapes=[
            pltpu.VMEM((N, N), jnp.bfloat16),
            pltpu.VMEM((N, D), jnp.bfloat16),
        ],
        compiler_params=pltpu.CompilerParams(
            dimension_semantics=("arbitrary", "arbitrary")),
    )(L, xb, w02, w1, w2x2)
    return out


def kernel(x, k, L, weight):
    return _graph_conv(x, k, L, weight)


# speedup vs baseline: 1.1283x; 1.0094x over previous
import functools

import jax
import jax.numpy as jnp
from jax.experimental import pallas as pl
from jax.experimental.pallas import tpu as pltpu

N = 4096
D = 256
BM = 512
N_BM = N // BM


def _body(l_ref, xb_ref, w02_ref, w1_ref, w2x2_ref, out_ref,
          lb_ref, yb_ref, part_ref):
    p = pl.program_id(0)
    i = pl.program_id(1)
    rows = pl.ds(i * BM, BM)

    @pl.when(p == 0)
    def _phase0():
        l_blk = l_ref[...].astype(jnp.bfloat16)
        lb_ref[rows, :] = l_blk
        y = jnp.dot(l_blk, xb_ref[...], preferred_element_type=jnp.float32)
        yb_ref[rows, :] = y.astype(jnp.bfloat16)
        part = (
            jnp.dot(xb_ref[rows, :], w02_ref[...],
                    preferred_element_type=jnp.float32)
            + jnp.dot(y, w1_ref[...], preferred_element_type=jnp.float32)
        )
        part_ref[rows, :] = part.astype(jnp.bfloat16)

    @pl.when(p == 1)
    def _phase1():
        z = jnp.dot(lb_ref[rows, :], yb_ref[...],
                    preferred_element_type=jnp.float32)
        out_ref[...] = part_ref[rows, :].astype(jnp.float32) + jnp.dot(
            z, w2x2_ref[...], preferred_element_type=jnp.float32)


@functools.partial(jax.jit, static_argnames=())
def _graph_conv(x, k, L, weight):
    scale = jnp.asarray(k, jnp.float32) / jnp.float32(weight.shape[0])
    w0 = weight[0] * scale
    w1 = weight[1] * scale
    w2 = weight[2] * scale
    w02 = w0 - w2
    w2x2 = 2.0 * w2
    xb = x.astype(jnp.bfloat16)

    grid = (2, N_BM)
    l_spec = pl.BlockSpec(
        (BM, N), lambda p, i: (jnp.where(p == 0, i, N_BM - 1), 0))
    full_spec = pl.BlockSpec((N, D), lambda p, i: (0, 0))
    out_spec = pl.BlockSpec(
        (BM, D), lambda p, i: (jnp.where(p == 0, 0, i), 0))
    w_spec = pl.BlockSpec((D, D), lambda p, i: (0, 0))

    out = pl.pallas_call(
        _body,
        grid=grid,
        in_specs=[l_spec, full_spec, w_spec, w_spec, w_spec],
        out_specs=out_spec,
        out_shape=jax.ShapeDtypeStruct((N, D), jnp.float32),
        scratch_shapes=[
            pltpu.VMEM((N, N), jnp.bfloat16),
            pltpu.VMEM((N, D), jnp.bfloat16),
            pltpu.VMEM((N, D), jnp.bfloat16),
        ],
        compiler_params=pltpu.CompilerParams(
            dimension_semantics=("arbitrary", "arbitrary")),
    )(L, xb, w02, w1, w2x2)
    return out


def kernel(x, k, L, weight):
    return _graph_conv(x, k, L, weight)
